# Initial kernel scaffold; baseline (speedup 1.0000x reference)
import jax
import jax.numpy as jnp
from jax.experimental import pallas as pl


def kernel(x, pos_table):
    raise NotImplementedError("write your pallas kernel here")


# speedup vs baseline: 1.6797x; 1.6797x over previous
import jax
import jax.numpy as jnp
from jax.experimental import pallas as pl


_TILE_S = 512


def _add_kernel(x_ref, pos_ref, o_ref):
    o_ref[...] = x_ref[...] + pos_ref[...]


def kernel(x, pos_table):
    B, S, D = x.shape
    grid = (S // _TILE_S, B)
    return pl.pallas_call(
        _add_kernel,
        grid=grid,
        in_specs=[
            pl.BlockSpec((1, _TILE_S, D), lambda s, b: (b, s, 0)),
            pl.BlockSpec((_TILE_S, D), lambda s, b: (s, 0)),
        ],
        out_specs=pl.BlockSpec((1, _TILE_S, D), lambda s, b: (b, s, 0)),
        out_shape=jax.ShapeDtypeStruct(x.shape, x.dtype),
    )(x, pos_table)


# speedup vs baseline: 1.8848x; 1.1221x over previous
import jax
import jax.numpy as jnp
from jax.experimental import pallas as pl


_TILE_S = 1024


def _add_kernel(x_ref, pos_ref, o_ref):
    o_ref[...] = x_ref[...] + pos_ref[...]


def kernel(x, pos_table):
    B, S, D = x.shape
    grid = (S // _TILE_S, B)
    return pl.pallas_call(
        _add_kernel,
        grid=grid,
        in_specs=[
            pl.BlockSpec((1, _TILE_S, D), lambda s, b: (b, s, 0)),
            pl.BlockSpec((_TILE_S, D), lambda s, b: (s, 0)),
        ],
        out_specs=pl.BlockSpec((1, _TILE_S, D), lambda s, b: (b, s, 0)),
        out_shape=jax.ShapeDtypeStruct(x.shape, x.dtype),
    )(x, pos_table)


# speedup vs baseline: 1.9986x; 1.0604x over previous
import jax
import jax.numpy as jnp
from jax.experimental import pallas as pl


_TILE_S = 2048


def _add_kernel(x_ref, pos_ref, o_ref):
    o_ref[...] = x_ref[...] + pos_ref[...]


def kernel(x, pos_table):
    B, S, D = x.shape
    grid = (S // _TILE_S, B)
    return pl.pallas_call(
        _add_kernel,
        grid=grid,
        in_specs=[
            pl.BlockSpec((1, _TILE_S, D), lambda s, b: (b, s, 0)),
            pl.BlockSpec((_TILE_S, D), lambda s, b: (s, 0)),
        ],
        out_specs=pl.BlockSpec((1, _TILE_S, D), lambda s, b: (b, s, 0)),
        out_shape=jax.ShapeDtypeStruct(x.shape, x.dtype),
    )(x, pos_table)
